# baseline (speedup 1.0000x reference)
import functools

import jax
import jax.numpy as jnp
from jax import lax
from jax.experimental import pallas as pl
from jax.experimental.pallas import tpu as pltpu
from jax.experimental.pallas import tpu_sc as plsc

_N = 10000
_E = 320000
_D = 128
_DE = 16
_T = 3

_NC = 2
_NS = 16
_NW = _NC * _NS
_CH = 128
_CPW = 80
_EPAD = _NW * _CPW * _CH
_NPAD = 10240
_RPW = _NPAD // _NS

_mesh = plsc.VectorSubcoreMesh(core_axis_name="c", subcore_axis_name="s")


@functools.partial(
    pl.kernel,
    mesh=_mesh,
    out_type=jax.ShapeDtypeStruct((_NC, _NPAD, _D), jnp.float32),
    scratch_types=[
        pltpu.VMEM((_CPW, _CH), jnp.int32),
        pltpu.VMEM((_CPW, _CH), jnp.int32),
        pltpu.VMEM((_CH, _D), jnp.float32),
        pltpu.VMEM_SHARED((_NPAD, _D), jnp.float32),
        pltpu.SemaphoreType.DMA,
    ],
)
def _sc_spmm(h_hbm, widx_hbm, vidx_hbm, zeros_hbm, out_hbm,
             widx_v, vidx_v, rows_v, acc, sem):
    c = lax.axis_index("c")
    s = lax.axis_index("s")
    wid = s * _NC + c
    pltpu.sync_copy(widx_hbm.at[wid], widx_v)
    pltpu.sync_copy(vidx_hbm.at[wid], vidx_v)
    pltpu.sync_copy(zeros_hbm, acc.at[pl.ds(s * _RPW, _RPW)])
    plsc.subcore_barrier()

    def body(j, carry):
        pltpu.async_copy(h_hbm.at[widx_v.at[j]], rows_v, sem).wait()
        pltpu.sync_copy(rows_v, acc.at[vidx_v.at[j]], add=True)
        return carry

    lax.fori_loop(0, _CPW, body, 0)
    plsc.subcore_barrier()
    pltpu.sync_copy(acc.at[pl.ds(s * _RPW, _RPW)],
                    out_hbm.at[c, pl.ds(s * _RPW, _RPW)])


@functools.partial(
    pl.kernel,
    mesh=_mesh,
    out_type=jax.ShapeDtypeStruct((_NC, _NPAD, _D), jnp.float32),
    scratch_types=[
        pltpu.VMEM((_CPW, _CH), jnp.int32),
        pltpu.VMEM((_CH, _D), jnp.float32),
        pltpu.VMEM_SHARED((_NPAD, _D), jnp.float32),
        pltpu.SemaphoreType.DMA,
    ],
)
def _sc_aux(eaug_hbm, vidx_hbm, zeros_hbm, out_hbm,
            vidx_v, rows_v, acc, sem):
    c = lax.axis_index("c")
    s = lax.axis_index("s")
    wid = s * _NC + c
    base = wid * _CPW * _CH
    pltpu.sync_copy(vidx_hbm.at[wid], vidx_v)
    pltpu.sync_copy(zeros_hbm, acc.at[pl.ds(s * _RPW, _RPW)])
    plsc.subcore_barrier()

    def body(j, carry):
        pltpu.async_copy(eaug_hbm.at[pl.ds(base + j * _CH, _CH)], rows_v,
                         sem).wait()
        pltpu.sync_copy(rows_v, acc.at[vidx_v.at[j]], add=True)
        return carry

    lax.fori_loop(0, _CPW, body, 0)
    plsc.subcore_barrier()
    pltpu.sync_copy(acc.at[pl.ds(s * _RPW, _RPW)],
                    out_hbm.at[c, pl.ds(s * _RPW, _RPW)])


_BN = 1000


def _tc_round_body(h_ref, pp_ref, sa_ref, u1_ref, u2_ref, waug_ref,
                   m1_ref, m2_ref, mb_ref, out_ref):
    hb = h_ref[...]
    p = pp_ref[0] + pp_ref[1]
    sa = sa_ref[0] + sa_ref[1]
    deg = sa[:, _DE:_DE + 1]
    agg = (deg * jnp.dot(hb, u1_ref[...], preferred_element_type=jnp.float32)
           + jnp.dot(p, u2_ref[...], preferred_element_type=jnp.float32)
           + jnp.dot(sa, waug_ref[...], preferred_element_type=jnp.float32))
    out = (jnp.dot(hb, m1_ref[...], preferred_element_type=jnp.float32)
           + jnp.dot(agg, m2_ref[...], preferred_element_type=jnp.float32)
           + mb_ref[...])
    out_ref[...] = jnp.maximum(out, 0.0)


_tc_round = pl.pallas_call(
    _tc_round_body,
    grid=(_N // _BN,),
    in_specs=[
        pl.BlockSpec((_BN, _D), lambda i: (i, 0)),
        pl.BlockSpec((_NC, _BN, _D), lambda i: (0, i, 0)),
        pl.BlockSpec((_NC, _BN, _D), lambda i: (0, i, 0)),
        pl.BlockSpec((_D, _D), lambda i: (0, 0)),
        pl.BlockSpec((_D, _D), lambda i: (0, 0)),
        pl.BlockSpec((_D, _D), lambda i: (0, 0)),
        pl.BlockSpec((_D, _D), lambda i: (0, 0)),
        pl.BlockSpec((_D, _D), lambda i: (0, 0)),
        pl.BlockSpec((1, _D), lambda i: (0, 0)),
    ],
    out_specs=pl.BlockSpec((_BN, _D), lambda i: (i, 0)),
    out_shape=jax.ShapeDtypeStruct((_N, _D), jnp.float32),
)


def kernel(node_features, edge_index, edge_attr, U_w, U_b, M_w, M_b):
    v = edge_index[0]
    w = edge_index[1]
    npad = _EPAD - _E
    dummy = _N + (jnp.arange(npad, dtype=jnp.int32) % (_NPAD - _N))
    vpad = jnp.concatenate([v, dummy])
    wpad = jnp.concatenate([w, jnp.zeros((npad,), jnp.int32)])
    widx = wpad.reshape(_NW, _CPW, _CH)
    vidx = vpad.reshape(_NW, _CPW, _CH)

    eaug = jnp.concatenate(
        [edge_attr,
         jnp.ones((_E, 1), jnp.float32),
         jnp.zeros((_E, _D - _DE - 1), jnp.float32)], axis=1)
    eaug = jnp.concatenate([eaug, jnp.zeros((npad, _D), jnp.float32)], axis=0)

    zeros_d = jnp.zeros((_RPW, _D), jnp.float32)

    sa = _sc_aux(eaug, vidx, zeros_d)

    h = node_features
    for k in range(_T):
        u1 = U_w[k, :_D]
        u2 = U_w[k, _D:2 * _D]
        waug = jnp.concatenate(
            [U_w[k, 2 * _D:], U_b[k][None, :],
             jnp.zeros((_D - _DE - 1, _D), jnp.float32)], axis=0)
        pp = _sc_spmm(h, widx, vidx, zeros_d)
        h = _tc_round(h, pp, sa, u1, u2, waug,
                      M_w[k, :_D], M_w[k, _D:], M_b[k][None, :])
    return (edge_attr, h)


# speedup vs baseline: 1.4507x; 1.4507x over previous
import functools

import jax
import jax.numpy as jnp
from jax import lax
from jax.experimental import pallas as pl
from jax.experimental.pallas import tpu as pltpu
from jax.experimental.pallas import tpu_sc as plsc

_N = 10000
_E = 320000
_D = 128
_DE = 16
_T = 3

_NC = 2
_NS = 16
_NW = _NC * _NS
_CH = 128
_CPW = 79
_EPAD = _NW * _CPW * _CH
_NPAD = 10240
_RPW = _NPAD // _NS

_mesh = plsc.VectorSubcoreMesh(core_axis_name="c", subcore_axis_name="s")


@functools.partial(
    pl.kernel,
    mesh=_mesh,
    out_type=jax.ShapeDtypeStruct((_NC, _NPAD, _D), jnp.float32),
    scratch_types=[
        pltpu.VMEM((_CPW, _CH), jnp.int32),
        pltpu.VMEM((_CPW, _CH), jnp.int32),
        pltpu.VMEM((_CH, _D), jnp.float32),
        pltpu.VMEM_SHARED((_NPAD, _D), jnp.float32),
        pltpu.SemaphoreType.DMA,
    ],
)
def _sc_spmm(h_hbm, widx_hbm, vidx_hbm, zeros_hbm, out_hbm,
             widx_v, vidx_v, rows_v, acc, sem):
    c = lax.axis_index("c")
    s = lax.axis_index("s")
    wid = s * _NC + c
    pltpu.sync_copy(widx_hbm.at[wid], widx_v)
    pltpu.sync_copy(vidx_hbm.at[wid], vidx_v)
    pltpu.sync_copy(zeros_hbm, acc.at[pl.ds(s * _RPW, _RPW)])
    plsc.subcore_barrier()

    def body(j, carry):
        pltpu.async_copy(h_hbm.at[widx_v.at[j]], rows_v, sem).wait()
        pltpu.sync_copy(rows_v, acc.at[vidx_v.at[j]], add=True)
        return carry

    lax.fori_loop(0, _CPW, body, 0)
    plsc.subcore_barrier()
    pltpu.sync_copy(acc.at[pl.ds(s * _RPW, _RPW)],
                    out_hbm.at[c, pl.ds(s * _RPW, _RPW)])


@functools.partial(
    pl.kernel,
    mesh=_mesh,
    out_type=jax.ShapeDtypeStruct((_NC, _NPAD, _D), jnp.float32),
    scratch_types=[
        pltpu.VMEM((_CPW, _CH), jnp.int32),
        pltpu.VMEM((_CH, _D), jnp.float32),
        pltpu.VMEM_SHARED((_NPAD, _D), jnp.float32),
        pltpu.SemaphoreType.DMA,
    ],
)
def _sc_aux(eaug_hbm, vidx_hbm, zeros_hbm, out_hbm,
            vidx_v, rows_v, acc, sem):
    c = lax.axis_index("c")
    s = lax.axis_index("s")
    wid = s * _NC + c
    base = wid * _CPW * _CH
    pltpu.sync_copy(vidx_hbm.at[wid], vidx_v)
    pltpu.sync_copy(zeros_hbm, acc.at[pl.ds(s * _RPW, _RPW)])
    plsc.subcore_barrier()

    def body(j, carry):
        pltpu.async_copy(eaug_hbm.at[pl.ds(base + j * _CH, _CH)], rows_v,
                         sem).wait()
        pltpu.sync_copy(rows_v, acc.at[vidx_v.at[j]], add=True)
        return carry

    lax.fori_loop(0, _CPW, body, 0)
    plsc.subcore_barrier()
    pltpu.sync_copy(acc.at[pl.ds(s * _RPW, _RPW)],
                    out_hbm.at[c, pl.ds(s * _RPW, _RPW)])


_BN = 1000


def _tc_round_body(h_ref, pp_ref, sa_ref, u1_ref, u2_ref, waug_ref,
                   m1_ref, m2_ref, mb_ref, out_ref):
    hb = h_ref[...]
    p = pp_ref[0] + pp_ref[1]
    sa = sa_ref[0] + sa_ref[1]
    deg = sa[:, _DE:_DE + 1]
    agg = (deg * jnp.dot(hb, u1_ref[...], preferred_element_type=jnp.float32)
           + jnp.dot(p, u2_ref[...], preferred_element_type=jnp.float32)
           + jnp.dot(sa, waug_ref[...], preferred_element_type=jnp.float32))
    out = (jnp.dot(hb, m1_ref[...], preferred_element_type=jnp.float32)
           + jnp.dot(agg, m2_ref[...], preferred_element_type=jnp.float32)
           + mb_ref[...])
    out_ref[...] = jnp.maximum(out, 0.0)


_tc_round = pl.pallas_call(
    _tc_round_body,
    grid=(_N // _BN,),
    in_specs=[
        pl.BlockSpec((_BN, _D), lambda i: (i, 0)),
        pl.BlockSpec((_NC, _BN, _D), lambda i: (0, i, 0)),
        pl.BlockSpec((_NC, _BN, _D), lambda i: (0, i, 0)),
        pl.BlockSpec((_D, _D), lambda i: (0, 0)),
        pl.BlockSpec((_D, _D), lambda i: (0, 0)),
        pl.BlockSpec((_D, _D), lambda i: (0, 0)),
        pl.BlockSpec((_D, _D), lambda i: (0, 0)),
        pl.BlockSpec((_D, _D), lambda i: (0, 0)),
        pl.BlockSpec((1, _D), lambda i: (0, 0)),
    ],
    out_specs=pl.BlockSpec((_BN, _D), lambda i: (i, 0)),
    out_shape=jax.ShapeDtypeStruct((_N, _D), jnp.float32),
)


def kernel(node_features, edge_index, edge_attr, U_w, U_b, M_w, M_b):
    v = edge_index[0]
    w = edge_index[1]
    npad = _EPAD - _E
    dummy = _N + (jnp.arange(npad, dtype=jnp.int32) % (_NPAD - _N))
    vpad = jnp.concatenate([v, dummy])
    wpad = jnp.concatenate([w, jnp.zeros((npad,), jnp.int32)])
    widx = wpad.reshape(_NW, _CPW, _CH)
    vidx = vpad.reshape(_NW, _CPW, _CH)

    eaug = jnp.concatenate(
        [edge_attr,
         jnp.ones((_E, 1), jnp.float32),
         jnp.zeros((_E, _D - _DE - 1), jnp.float32)], axis=1)
    eaug = jnp.concatenate([eaug, jnp.zeros((npad, _D), jnp.float32)], axis=0)

    zeros_d = jnp.zeros((_RPW, _D), jnp.float32)

    sa = _sc_aux(eaug, vidx, zeros_d)

    h = node_features
    for k in range(_T):
        u1 = U_w[k, :_D]
        u2 = U_w[k, _D:2 * _D]
        waug = jnp.concatenate(
            [U_w[k, 2 * _D:], U_b[k][None, :],
             jnp.zeros((_D - _DE - 1, _D), jnp.float32)], axis=0)
        pp = _sc_spmm(h, widx, vidx, zeros_d)
        h = _tc_round(h, pp, sa, u1, u2, waug,
                      M_w[k, :_D], M_w[k, _D:], M_b[k][None, :])
    return (edge_attr, h)


# speedup vs baseline: 1.4635x; 1.0088x over previous
import functools

import jax
import jax.numpy as jnp
from jax import lax
from jax.experimental import pallas as pl
from jax.experimental.pallas import tpu as pltpu
from jax.experimental.pallas import tpu_sc as plsc

_N = 10000
_E = 320000
_D = 128
_DE = 16
_T = 3

_NC = 2
_NS = 16
_NW = _NC * _NS
_CH = 128
_CPW = 79
_EPAD = _NW * _CPW * _CH
_NPAD = 10240
_RPW = _NPAD // _NS

_mesh = plsc.VectorSubcoreMesh(core_axis_name="c", subcore_axis_name="s")


@functools.partial(
    pl.kernel,
    mesh=_mesh,
    out_type=jax.ShapeDtypeStruct((_NC, _NPAD, _D), jnp.float32),
    scratch_types=[
        pltpu.VMEM((_CPW, _CH), jnp.int32),
        pltpu.VMEM((_CPW, _CH), jnp.int32),
        pltpu.VMEM((_CH, _D), jnp.float32),
        pltpu.VMEM_SHARED((_NPAD, _D), jnp.float32),
        pltpu.SemaphoreType.DMA,
    ],
)
def _sc_spmm(h_hbm, widx_hbm, vidx_hbm, zeros_hbm, out_hbm,
             widx_v, vidx_v, rows_v, acc, sem):
    c = lax.axis_index("c")
    s = lax.axis_index("s")
    wid = s * _NC + c
    pltpu.sync_copy(widx_hbm.at[wid], widx_v)
    pltpu.sync_copy(vidx_hbm.at[wid], vidx_v)
    pltpu.sync_copy(zeros_hbm, acc.at[pl.ds(s * _RPW, _RPW)])
    plsc.subcore_barrier()

    def body(j, carry):
        pltpu.async_copy(h_hbm.at[widx_v.at[j]], rows_v, sem).wait()
        pltpu.sync_copy(rows_v, acc.at[vidx_v.at[j]], add=True)
        return carry

    lax.fori_loop(0, _CPW, body, 0)
    plsc.subcore_barrier()
    pltpu.sync_copy(acc.at[pl.ds(s * _RPW, _RPW)],
                    out_hbm.at[c, pl.ds(s * _RPW, _RPW)])


_BN = 1000


def _tc_round_body(h_ref, pp_ref, sa_ref, u1_ref, u2_ref, waug_ref,
                   m1_ref, m2_ref, mb_ref, out_ref):
    hb = h_ref[...]
    p = pp_ref[0] + pp_ref[1]
    sa = sa_ref[0] + sa_ref[1]
    deg = sa[:, _DE:_DE + 1]
    agg = (deg * jnp.dot(hb, u1_ref[...], preferred_element_type=jnp.float32)
           + jnp.dot(p, u2_ref[...], preferred_element_type=jnp.float32)
           + jnp.dot(sa, waug_ref[...], preferred_element_type=jnp.float32))
    out = (jnp.dot(hb, m1_ref[...], preferred_element_type=jnp.float32)
           + jnp.dot(agg, m2_ref[...], preferred_element_type=jnp.float32)
           + mb_ref[...])
    out_ref[...] = jnp.maximum(out, 0.0)


_tc_round = pl.pallas_call(
    _tc_round_body,
    grid=(_N // _BN,),
    in_specs=[
        pl.BlockSpec((_BN, _D), lambda i: (i, 0)),
        pl.BlockSpec((_NC, _BN, _D), lambda i: (0, i, 0)),
        pl.BlockSpec((_NC, _BN, _D), lambda i: (0, i, 0)),
        pl.BlockSpec((_D, _D), lambda i: (0, 0)),
        pl.BlockSpec((_D, _D), lambda i: (0, 0)),
        pl.BlockSpec((_D, _D), lambda i: (0, 0)),
        pl.BlockSpec((_D, _D), lambda i: (0, 0)),
        pl.BlockSpec((_D, _D), lambda i: (0, 0)),
        pl.BlockSpec((1, _D), lambda i: (0, 0)),
    ],
    out_specs=pl.BlockSpec((_BN, _D), lambda i: (i, 0)),
    out_shape=jax.ShapeDtypeStruct((_N, _D), jnp.float32),
)


def kernel(node_features, edge_index, edge_attr, U_w, U_b, M_w, M_b):
    v = edge_index[0]
    w = edge_index[1]
    npad = _EPAD - _E
    dummy = _N + (jnp.arange(npad, dtype=jnp.int32) % (_NPAD - _N))
    vpad = jnp.concatenate([v, dummy])
    wpad = jnp.concatenate([w, jnp.zeros((npad,), jnp.int32)])
    widx = wpad.reshape(_NW, _CPW, _CH)
    vidx = vpad.reshape(_NW, _CPW, _CH)

    eaug = jnp.concatenate(
        [edge_attr,
         jnp.ones((_E, 1), jnp.float32),
         jnp.zeros((_E, _D - _DE - 1), jnp.float32)], axis=1)
    eaug = jnp.concatenate([eaug, jnp.zeros((npad, _D), jnp.float32)], axis=0)

    lin_idx = jnp.arange(_EPAD, dtype=jnp.int32).reshape(_NW, _CPW, _CH)
    zeros_d = jnp.zeros((_RPW, _D), jnp.float32)

    sa = _sc_spmm(eaug, lin_idx, vidx, zeros_d)

    h = node_features
    for k in range(_T):
        u1 = U_w[k, :_D]
        u2 = U_w[k, _D:2 * _D]
        waug = jnp.concatenate(
            [U_w[k, 2 * _D:], U_b[k][None, :],
             jnp.zeros((_D - _DE - 1, _D), jnp.float32)], axis=0)
        pp = _sc_spmm(h, widx, vidx, zeros_d)
        h = _tc_round(h, pp, sa, u1, u2, waug,
                      M_w[k, :_D], M_w[k, _D:], M_b[k][None, :])
    return (edge_attr, h)
